# baseline (speedup 1.0000x reference)
import jax
import jax.numpy as jnp
from jax.experimental import pallas as pl
from jax.experimental.pallas import tpu as pltpu

_LQ = 200
_BBL = 512


def _rosa_block(u_ref, x_ref, v_ref, o_ref):
    xq = x_ref[...]
    vq = v_ref[...]
    u = u_ref[0, 0]

    rows = jax.lax.broadcasted_iota(jnp.int32, (_LQ, 1), 0)
    out = jnp.full((_LQ, _BBL), u, dtype=jnp.float32)
    o_ref[...] = out
    for tp in range(_LQ - 1):
        lo = ((tp + 1) // 8) * 8
        xc = xq[tp:tp + 1, :]
        vc = vq[tp:tp + 1, :]
        m = (xq[lo:, :] == xc) & (rows[lo:, :] > tp)
        o_ref[lo:, :] = jnp.where(m, vc, o_ref[lo:, :])


def kernel(x, v, u):
    B, L = x.shape
    xT = x.astype(jnp.int32).T
    vT = v.T
    u_arr = jnp.full((1, 1), u, dtype=jnp.float32)

    out = pl.pallas_call(
        _rosa_block,
        grid=(B // _BBL,),
        in_specs=[
            pl.BlockSpec(memory_space=pltpu.SMEM),
            pl.BlockSpec((L, _BBL), lambda i: (0, i)),
            pl.BlockSpec((L, _BBL), lambda i: (0, i)),
        ],
        out_specs=pl.BlockSpec((L, _BBL), lambda i: (0, i)),
        out_shape=jax.ShapeDtypeStruct((L, B), jnp.float32),
    )(u_arr, xT, vT)
    return out.T


# speedup vs baseline: 1.3871x; 1.3871x over previous
import jax
import jax.numpy as jnp
from jax.experimental import pallas as pl
from jax.experimental.pallas import tpu as pltpu

_LQ = 200
_BBL = 128


def _rosa_block(u_ref, x_ref, v_ref, o_ref):
    xq = x_ref[...]
    vq = v_ref[...]
    u = u_ref[0, 0]

    rows = jax.lax.broadcasted_iota(jnp.int32, (_LQ, 1), 0)
    out = jnp.full((_LQ, _BBL), u, dtype=jnp.float32)
    o_ref[...] = out
    for tp in range(_LQ - 1):
        lo = ((tp + 1) // 8) * 8
        xc = xq[tp:tp + 1, :]
        vc = vq[tp:tp + 1, :]
        m = (xq[lo:, :] == xc) & (rows[lo:, :] > tp)
        o_ref[lo:, :] = jnp.where(m, vc, o_ref[lo:, :])


def kernel(x, v, u):
    B, L = x.shape
    xT = x.astype(jnp.int32).T
    vT = v.T
    u_arr = jnp.full((1, 1), u, dtype=jnp.float32)

    out = pl.pallas_call(
        _rosa_block,
        grid=(B // _BBL,),
        in_specs=[
            pl.BlockSpec(memory_space=pltpu.SMEM),
            pl.BlockSpec((L, _BBL), lambda i: (0, i)),
            pl.BlockSpec((L, _BBL), lambda i: (0, i)),
        ],
        out_specs=pl.BlockSpec((L, _BBL), lambda i: (0, i)),
        out_shape=jax.ShapeDtypeStruct((L, B), jnp.float32),
    )(u_arr, xT, vT)
    return out.T
